# Initial kernel scaffold; baseline (speedup 1.0000x reference)
import jax
import jax.numpy as jnp
from jax.experimental import pallas as pl


def kernel(points, embeddings, leaf_mask, W1, b1, W2, b2):
    raise NotImplementedError("write your pallas kernel here")


# speedup vs baseline: 1.6335x; 1.6335x over previous
import functools

import jax
import jax.numpy as jnp
from jax.experimental import pallas as pl


def _body(rows_ref, emb_t_ref, cols_ref, embc_ref, W1_ref, b1_ref, W2_ref,
          b2_ref, out_ref):
    rows = rows_ref[0]
    emb_t = emb_t_ref[0]
    cols = cols_ref[0]
    embc = embc_ref[0]

    px_row = rows[0:1, :]
    py_row = rows[1:2, :]
    mask_row = rows[2:3, :] > 0.0
    px_col = cols[:, 0:1]
    py_col = cols[:, 1:2]
    mask_col = cols[:, 2:3] > 0.0

    dx = px_row - px_col
    dy = py_row - py_col
    dist = jnp.sqrt(dx * dx + dy * dy)
    neighbor = (dist < 0.03) & mask_row
    n_count = jnp.sum(neighbor.astype(jnp.float32), axis=1, keepdims=True)

    norm_row = jnp.sqrt(jnp.sum(emb_t * emb_t, axis=0, keepdims=True))
    norm_col = jnp.sqrt(jnp.sum(embc * embc, axis=1, keepdims=True))
    gram = jax.lax.dot_general(embc, emb_t, (((1,), (0,)), ((), ())))
    denom = jnp.maximum(norm_col, 1e-8) * jnp.maximum(norm_row, 1e-8)
    sims = gram / denom

    similar = neighbor & (sims > 0.7)
    similar_f = similar.astype(jnp.float32)
    cnt_sim = jnp.sum(similar_f, axis=1, keepdims=True)
    mean_sim = jax.lax.dot_general(
        similar_f, emb_t, (((1,), (1,)), ((), ()))) / jnp.maximum(cnt_sim, 1.0)

    combined = jnp.concatenate([embc, mean_sim], axis=1)
    h = jnp.maximum(combined @ W1_ref[...] + b1_ref[...], 0.0)
    out = h @ W2_ref[...] + b2_ref[...]

    update = mask_col & (n_count > 1.0) & (cnt_sim > 0.0)
    refined = jnp.where(update, out, embc)
    leaf_count = jnp.sum(rows[2:3, :], axis=1, keepdims=True)
    out_ref[0] = jnp.where(leaf_count < 10.0, embc, refined)


@jax.jit
def kernel(points, embeddings, leaf_mask, W1, b1, W2, b2):
    B, N, D = embeddings.shape
    TI = 256

    mask_f = leaf_mask.astype(jnp.float32)
    rows = jnp.concatenate(
        [jnp.transpose(points, (0, 2, 1)), mask_f[:, None, :],
         jnp.zeros((B, 5, N), jnp.float32)], axis=1)
    cols = jnp.concatenate(
        [points, mask_f[:, :, None], jnp.zeros((B, N, 5), jnp.float32)],
        axis=2)
    emb_t = jnp.transpose(embeddings, (0, 2, 1))

    grid = (B, N // TI)
    return pl.pallas_call(
        _body,
        grid=grid,
        in_specs=[
            pl.BlockSpec((1, 8, N), lambda b, i: (b, 0, 0)),
            pl.BlockSpec((1, D, N), lambda b, i: (b, 0, 0)),
            pl.BlockSpec((1, TI, 8), lambda b, i: (b, i, 0)),
            pl.BlockSpec((1, TI, D), lambda b, i: (b, i, 0)),
            pl.BlockSpec((2 * D, D), lambda b, i: (0, 0)),
            pl.BlockSpec((1, D), lambda b, i: (0, 0)),
            pl.BlockSpec((D, D), lambda b, i: (0, 0)),
            pl.BlockSpec((1, D), lambda b, i: (0, 0)),
        ],
        out_specs=pl.BlockSpec((1, TI, D), lambda b, i: (b, i, 0)),
        out_shape=jax.ShapeDtypeStruct((B, N, D), jnp.float32),
    )(rows, emb_t, cols, embeddings, W1, b1.reshape(1, D), W2,
      b2.reshape(1, D))


# speedup vs baseline: 1.9946x; 1.2211x over previous
import numpy as np

import jax
import jax.numpy as jnp
from jax.experimental import pallas as pl


def _dist2_threshold() -> np.float32:
    c = np.float32(0.03)
    t = np.float32(np.float64(c) * np.float64(c))
    while np.sqrt(t) >= c:
        t = np.nextafter(t, np.float32(0.0), dtype=np.float32)
    while np.sqrt(t) < c:
        t = np.nextafter(t, np.float32(np.inf), dtype=np.float32)
    return t


DIST2_THRESH = float(_dist2_threshold())


def _body(rows_ref, embx_ref, cols_ref, embc_ref, W1_ref, b1_ref, W2_ref,
          b2_ref, out_ref):
    rows = rows_ref[0]
    embx = embx_ref[0]
    cols = cols_ref[0]
    embc = embc_ref[0]
    emb_t = embx[0:64, :]
    ones_row = embx[64:65, :]

    px_row = rows[0:1, :]
    py_row = rows[1:2, :]
    mask_row = rows[2:3, :] > 0.0
    px_col = cols[:, 0:1]
    py_col = cols[:, 1:2]
    mask_col = cols[:, 2:3] > 0.0

    dx = px_row - px_col
    dy = py_row - py_col
    d2 = dx * dx + dy * dy
    neighbor = (d2 < DIST2_THRESH) & mask_row
    neighbor_f = neighbor.astype(jnp.float32)
    n_count = jax.lax.dot_general(
        neighbor_f, ones_row, (((1,), (1,)), ((), ())))

    norm_row = jnp.sqrt(jnp.sum(emb_t * emb_t, axis=0, keepdims=True))
    norm_col = jnp.sqrt(jnp.sum(embc * embc, axis=1, keepdims=True))
    gram = jax.lax.dot_general(embc, emb_t, (((1,), (0,)), ((), ())))
    thresh = (0.7 * jnp.maximum(norm_col, 1e-8)) * jnp.maximum(norm_row, 1e-8)

    similar_f = (neighbor & (gram > thresh)).astype(jnp.float32)
    acc = jax.lax.dot_general(
        similar_f, embx, (((1,), (1,)), ((), ())))
    cnt_sim = acc[:, 64:65]
    mean_sim = acc[:, 0:64] / jnp.maximum(cnt_sim, 1.0)

    combined = jnp.concatenate([embc, mean_sim], axis=1)
    h = jnp.maximum(combined @ W1_ref[...] + b1_ref[...], 0.0)
    out = h @ W2_ref[...] + b2_ref[...]

    update = mask_col & (n_count > 1.0) & (cnt_sim > 0.0)
    refined = jnp.where(update, out, embc)
    leaf_count = jnp.sum(rows[2:3, :], axis=1, keepdims=True)
    out_ref[0] = jnp.where(leaf_count < 10.0, embc, refined)


@jax.jit
def kernel(points, embeddings, leaf_mask, W1, b1, W2, b2):
    B, N, D = embeddings.shape
    TI = 256

    mask_f = leaf_mask.astype(jnp.float32)
    rows = jnp.concatenate(
        [jnp.transpose(points, (0, 2, 1)), mask_f[:, None, :],
         jnp.zeros((B, 5, N), jnp.float32)], axis=1)
    cols = jnp.concatenate(
        [points, mask_f[:, :, None], jnp.zeros((B, N, 5), jnp.float32)],
        axis=2)
    embx = jnp.concatenate(
        [jnp.transpose(embeddings, (0, 2, 1)),
         jnp.ones((B, 1, N), jnp.float32),
         jnp.zeros((B, 7, N), jnp.float32)], axis=1)

    grid = (B, N // TI)
    return pl.pallas_call(
        _body,
        grid=grid,
        in_specs=[
            pl.BlockSpec((1, 8, N), lambda b, i: (b, 0, 0)),
            pl.BlockSpec((1, 72, N), lambda b, i: (b, 0, 0)),
            pl.BlockSpec((1, TI, 8), lambda b, i: (b, i, 0)),
            pl.BlockSpec((1, TI, D), lambda b, i: (b, i, 0)),
            pl.BlockSpec((2 * D, D), lambda b, i: (0, 0)),
            pl.BlockSpec((1, D), lambda b, i: (0, 0)),
            pl.BlockSpec((D, D), lambda b, i: (0, 0)),
            pl.BlockSpec((1, D), lambda b, i: (0, 0)),
        ],
        out_specs=pl.BlockSpec((1, TI, D), lambda b, i: (b, i, 0)),
        out_shape=jax.ShapeDtypeStruct((B, N, D), jnp.float32),
    )(rows, embx, cols, embeddings, W1, b1.reshape(1, D), W2,
      b2.reshape(1, D))
